# initial kernel scaffold (unmeasured)
import jax
import jax.numpy as jnp
from jax import lax
from jax.experimental import pallas as pl
from jax.experimental.pallas import tpu as pltpu


def kernel(
    x,
):
    def body(*refs):
        pass

    out_shape = jax.ShapeDtypeStruct(..., jnp.float32)
    return pl.pallas_call(body, out_shape=out_shape)(...)


# baseline (device time: 4802357 ns/iter reference)
import jax
import jax.numpy as jnp
from jax import lax
from jax.experimental import pallas as pl
from jax.experimental.pallas import tpu as pltpu

N_DEV = 8
M = 4096
N_TOTAL = 1024
BLK_N = 128
GRID = N_TOTAL // BLK_N

STEPS = [(2, 1), (4, 2), (4, 1), (8, 4), (8, 2), (8, 1)]


def kernel(x):
    def body(x_ref, out_ref, buf, comm, send_sem, recv_sem, ready_sems):
        my_i = lax.axis_index("i")

        def sweep(d, k, desc):
            xv = buf[...]
            idx = lax.broadcasted_iota(jnp.int32, xv.shape, 0)
            lower = (idx & d) == 0
            up = ((idx & k) == 0) != desc
            nd = pltpu.roll(xv, M - d, 0)
            nu = pltpu.roll(xv, d, 0)
            partner = jnp.where(lower, nd, nu)
            keep_min = lower == up
            buf[...] = jnp.where(
                keep_min, jnp.minimum(xv, partner), jnp.maximum(xv, partner)
            )

        buf[...] = x_ref[...]
        desc0 = (my_i & STEPS[0][1]) != 0

        def stage_body(a, _):
            k = lax.shift_left(2, a)

            def j_body(b, _):
                d = lax.shift_right_logical(k, b + 1)
                sweep(d, k, desc0)
                return 0

            lax.fori_loop(0, a + 1, j_body, 0)
            return 0

        lax.fori_loop(0, 12, stage_body, 0)

        def bitonic_merge(desc):
            def b_body(b, _):
                d = lax.shift_right_logical(M, b + 1)
                sweep(d, 2 * M, desc)
                return 0

            lax.fori_loop(0, 12, b_body, 0)

        for s, (k, j) in enumerate(STEPS):
            partner = my_i ^ j
            pl.semaphore_signal(
                ready_sems.at[s],
                inc=1,
                device_id=(partner,),
                device_id_type=pl.DeviceIdType.MESH,
            )
            pl.semaphore_wait(ready_sems.at[s], 1)
            rdma = pltpu.make_async_remote_copy(
                src_ref=buf,
                dst_ref=comm,
                send_sem=send_sem,
                recv_sem=recv_sem,
                device_id=(partner,),
                device_id_type=pl.DeviceIdType.MESH,
            )
            rdma.start()
            rdma.wait()

            mine = buf[...]
            theirs = comm[...]
            lower = (my_i & j) == 0
            up = (my_i & k) == 0
            keep_min = lower == up
            buf[...] = jnp.where(
                keep_min,
                jnp.minimum(mine, theirs),
                jnp.maximum(mine, theirs),
            )
            if s + 1 < len(STEPS):
                desc_next = (my_i & STEPS[s + 1][1]) != 0
            else:
                desc_next = my_i < 0
            bitonic_merge(desc_next)

        out_ref[...] = buf[...]

    xb = x.astype(jnp.bfloat16)
    return pl.pallas_call(
        body,
        grid=(GRID,),
        in_specs=[
            pl.BlockSpec((M, BLK_N), lambda g: (0, g), memory_space=pltpu.VMEM)
        ],
        out_specs=pl.BlockSpec(
            (M, BLK_N), lambda g: (0, g), memory_space=pltpu.VMEM
        ),
        out_shape=jax.ShapeDtypeStruct((M, N_TOTAL), jnp.bfloat16),
        scratch_shapes=[
            pltpu.VMEM((M, BLK_N), jnp.bfloat16),
            pltpu.VMEM((M, BLK_N), jnp.bfloat16),
            pltpu.SemaphoreType.DMA,
            pltpu.SemaphoreType.DMA,
            pltpu.SemaphoreType.REGULAR((len(STEPS),)),
        ],
    )(xb)


# device time: 3770236 ns/iter; 1.2738x vs baseline; 1.2738x over previous
import jax
import jax.numpy as jnp
from jax import lax
from jax.experimental import pallas as pl
from jax.experimental.pallas import tpu as pltpu

N_DEV = 8
M = 4096
N_TOTAL = 1024
BLK_N = 128
GRID = N_TOTAL // BLK_N

STEPS = [(2, 1), (4, 2), (4, 1), (8, 4), (8, 2), (8, 1)]


def kernel(x):
    def body(x_ref, out_ref, buf, comm, send_sem, recv_sem, ready_sems):
        my_i = lax.axis_index("i")

        def rows_iota(shape):
            return lax.broadcasted_iota(jnp.int32, (shape[0], 1), 0)

        def sweep_masked(d, k, desc):
            xv = buf[...]
            idx = rows_iota(xv.shape)
            lower = (idx & d) == 0
            up = ((idx & k) == 0) != desc
            nd = pltpu.roll(xv, M - d, 0)
            nu = pltpu.roll(xv, d, 0)
            partner = jnp.where(lower, nd, nu)
            keep_min = lower == up
            buf[...] = jnp.where(
                keep_min, jnp.minimum(xv, partner), jnp.maximum(xv, partner)
            )

        def sweep_uniform(d, desc):
            xv = buf[...]
            idx = rows_iota(xv.shape)
            lower = (idx & d) == 0
            nd = pltpu.roll(xv, M - d, 0)
            nu = pltpu.roll(xv, d, 0)
            partner = jnp.where(lower, nd, nu)
            keep_min = lower != desc
            buf[...] = jnp.where(
                keep_min, jnp.minimum(xv, partner), jnp.maximum(xv, partner)
            )

        buf[...] = x_ref[...]
        desc0 = (my_i & 1) != 0

        def stage_body(a, _):
            k = lax.shift_left(2, a)

            def j_body(b, _):
                d = lax.shift_right_logical(k, b + 1)
                sweep_masked(d, k, desc0)
                return 0

            lax.fori_loop(0, a + 1, j_body, 0)
            return 0

        lax.fori_loop(0, 12, stage_body, 0)

        def bitonic_merge(desc):
            def b_body(b, _):
                d = lax.shift_right_logical(M, b + 1)
                sweep_uniform(d, desc)
                return 0

            lax.fori_loop(0, 12, b_body, 0)

        for s, (kk, jj) in enumerate(STEPS):
            partner = my_i ^ jj
            pl.semaphore_signal(
                ready_sems.at[s],
                inc=1,
                device_id=(partner,),
                device_id_type=pl.DeviceIdType.MESH,
            )
            pl.semaphore_wait(ready_sems.at[s], 1)
            rdma = pltpu.make_async_remote_copy(
                src_ref=buf,
                dst_ref=comm,
                send_sem=send_sem,
                recv_sem=recv_sem,
                device_id=(partner,),
                device_id_type=pl.DeviceIdType.MESH,
            )
            rdma.start()
            rdma.wait()

            mine = buf[...]
            theirs = comm[...]
            lower = (my_i & jj) == 0
            up = (my_i & kk) == 0
            keep_min = lower == up
            buf[...] = jnp.where(
                keep_min,
                jnp.minimum(mine, theirs),
                jnp.maximum(mine, theirs),
            )
            if jj == 1:
                bitonic_merge((my_i & kk) != 0)

        out_ref[...] = buf[...]

    xb = x.astype(jnp.bfloat16)
    return pl.pallas_call(
        body,
        grid=(GRID,),
        in_specs=[
            pl.BlockSpec((M, BLK_N), lambda g: (0, g), memory_space=pltpu.VMEM)
        ],
        out_specs=pl.BlockSpec(
            (M, BLK_N), lambda g: (0, g), memory_space=pltpu.VMEM
        ),
        out_shape=jax.ShapeDtypeStruct((M, N_TOTAL), jnp.bfloat16),
        scratch_shapes=[
            pltpu.VMEM((M, BLK_N), jnp.bfloat16),
            pltpu.VMEM((M, BLK_N), jnp.bfloat16),
            pltpu.SemaphoreType.DMA,
            pltpu.SemaphoreType.DMA,
            pltpu.SemaphoreType.REGULAR((len(STEPS),)),
        ],
    )(xb)
